# baseline (speedup 1.0000x reference)
import functools

import jax
import jax.numpy as jnp
from jax import lax
from jax.experimental import pallas as pl
from jax.experimental.pallas import tpu as pltpu
from jax.experimental.pallas import tpu_sc as plsc

VOCAB = 1000
EMBED = 1000
BATCH = 1024
SEQ = 20

UNIT = 16
RING = 5


@functools.lru_cache(maxsize=None)
def _build(batch, seq, embed):
    info = plsc.get_sparse_core_info()
    nc, ns = info.num_cores, info.num_subcores
    nw = nc * ns
    bpw = batch // nw
    assert bpw * nw == batch
    rpw = bpw * seq
    nunit = rpw // UNIT
    assert nunit * UNIT == rpw
    ring_rows = RING * UNIT
    assert ring_rows % seq == 0

    mesh = plsc.VectorSubcoreMesh(core_axis_name="c", subcore_axis_name="s")

    @functools.partial(
        pl.kernel,
        mesh=mesh,
        out_type=jax.ShapeDtypeStruct((batch, seq, embed), jnp.float32),
        compiler_params=pltpu.CompilerParams(use_tc_tiling_on_sc=False),
        scratch_types=[
            pltpu.VMEM((rpw,), jnp.int32),
            pltpu.VMEM((ring_rows, embed), jnp.float32),
            [pltpu.SemaphoreType.DMA] * RING,
            [pltpu.SemaphoreType.DMA] * 2,
        ],
    )
    def emb(x_hbm, table_hbm, out_hbm, idx_v, ring_v, sg, sw):
        wid = lax.axis_index("s") * nc + lax.axis_index("c")
        b0 = wid * bpw
        pltpu.sync_copy(x_hbm.at[pl.ds(b0 * seq, rpw)], idx_v)

        def gather(u):
            return pltpu.async_copy(
                table_hbm.at[idx_v.at[pl.ds(u * UNIT, UNIT)]],
                ring_v.at[pl.ds((u % RING) * UNIT, UNIT)],
                sg[u % RING],
            )

        pend_g = [gather(u) for u in range(RING)]
        issued = RING
        waited_g = 0
        waited_w = -1
        pend_w = [None] * bpw
        for k in range(bpw):
            last_u = (seq * k + seq - 1) // UNIT
            while waited_g <= last_u:
                pend_g[waited_g].wait()
                waited_g += 1
            pend_w[k] = pltpu.async_copy(
                ring_v.at[pl.ds((seq * k) % ring_rows, seq)],
                out_hbm.at[b0 + k],
                sw[k % 2],
            )
            while issued < nunit:
                wt = (UNIT * (issued - RING) + UNIT - 1) // seq
                if wt > k:
                    break
                while waited_w < wt:
                    waited_w += 1
                    pend_w[waited_w].wait()
                pend_g.append(gather(issued))
                issued += 1
        for k2 in range(waited_w + 1, bpw):
            pend_w[k2].wait()

    return emb


def kernel(x, table):
    emb = _build(BATCH, SEQ, EMBED)
    out = emb(x.reshape(-1), table)
    return jnp.maximum(out, jnp.float32(-3.0e38))


# speedup vs baseline: 1.1959x; 1.1959x over previous
import functools

import jax
import jax.numpy as jnp
from jax import lax
from jax.experimental import pallas as pl
from jax.experimental.pallas import tpu as pltpu
from jax.experimental.pallas import tpu_sc as plsc

VOCAB = 1000
EMBED = 1000
BATCH = 1024
SEQ = 20

UNIT = 16
RING = 5


@functools.lru_cache(maxsize=None)
def _build(batch, seq, embed):
    info = plsc.get_sparse_core_info()
    nc, ns = info.num_cores, info.num_subcores
    nw = nc * ns
    bpw = batch // nw
    assert bpw * nw == batch
    rpw = bpw * seq
    nunit = rpw // UNIT
    assert nunit * UNIT == rpw
    ring_rows = RING * UNIT
    assert ring_rows % seq == 0

    mesh = plsc.VectorSubcoreMesh(core_axis_name="c", subcore_axis_name="s")

    @functools.partial(
        pl.kernel,
        mesh=mesh,
        out_type=jax.ShapeDtypeStruct((batch, 24, 1024), jnp.float32),
        compiler_params=pltpu.CompilerParams(use_tc_tiling_on_sc=False),
        scratch_types=[
            pltpu.VMEM((rpw,), jnp.int32),
            pltpu.VMEM((ring_rows, embed), jnp.float32),
            [pltpu.SemaphoreType.DMA] * RING,
            [pltpu.SemaphoreType.DMA] * 2,
        ],
    )
    def emb(x_hbm, table_hbm, out_hbm, idx_v, ring_v, sg, sw):
        wid = lax.axis_index("s") * nc + lax.axis_index("c")
        b0 = wid * bpw
        pltpu.sync_copy(x_hbm.at[pl.ds(b0 * seq, rpw)], idx_v)

        def gather(u):
            return pltpu.async_copy(
                table_hbm.at[idx_v.at[pl.ds(u * UNIT, UNIT)]],
                ring_v.at[pl.ds((u % RING) * UNIT, UNIT)],
                sg[u % RING],
            )

        pend_g = [gather(u) for u in range(RING)]
        issued = RING
        waited_g = 0
        waited_w = -1
        pend_w = [None] * bpw
        for k in range(bpw):
            last_u = (seq * k + seq - 1) // UNIT
            while waited_g <= last_u:
                pend_g[waited_g].wait()
                waited_g += 1
            pend_w[k] = pltpu.async_copy(
                ring_v.at[pl.ds((seq * k) % ring_rows, seq)],
                out_hbm.at[b0 + k, pl.ds(0, seq), pl.ds(0, embed)],
                sw[k % 2],
            )
            while issued < nunit:
                wt = (UNIT * (issued - RING) + UNIT - 1) // seq
                if wt > k:
                    break
                while waited_w < wt:
                    waited_w += 1
                    pend_w[waited_w].wait()
                pend_g.append(gather(issued))
                issued += 1
        for k2 in range(waited_w + 1, bpw):
            pend_w[k2].wait()

    return emb


def kernel(x, table):
    emb = _build(BATCH, SEQ, EMBED)
    out = emb(x.reshape(-1), table)
    return out[:, :SEQ, :EMBED]


# speedup vs baseline: 1.2693x; 1.0614x over previous
import functools

import jax
import jax.numpy as jnp
from jax import lax
from jax.experimental import pallas as pl
from jax.experimental.pallas import tpu as pltpu
from jax.experimental.pallas import tpu_sc as plsc

VOCAB = 1000
EMBED = 1000
BATCH = 1024
SEQ = 20

UNIT = 16
RING = 5


@functools.lru_cache(maxsize=None)
def _build(batch, seq, embed):
    info = plsc.get_sparse_core_info()
    nc, ns = info.num_cores, info.num_subcores
    nw = nc * ns
    bpw = batch // nw
    assert bpw * nw == batch
    rpw = bpw * seq
    nunit = rpw // UNIT
    assert nunit * UNIT == rpw
    ring_rows = RING * UNIT
    assert ring_rows % seq == 0

    mesh = plsc.VectorSubcoreMesh(core_axis_name="c", subcore_axis_name="s")

    @functools.partial(
        pl.kernel,
        mesh=mesh,
        out_type=jax.ShapeDtypeStruct((batch, seq, embed), jnp.float32),
        compiler_params=pltpu.CompilerParams(use_tc_tiling_on_sc=False),
        scratch_types=[
            pltpu.VMEM((rpw,), jnp.int32),
            pltpu.VMEM((ring_rows, embed), jnp.float32),
            [pltpu.SemaphoreType.DMA] * RING,
            [pltpu.SemaphoreType.DMA] * 2,
        ],
    )
    def emb(x_hbm, table_hbm, out_hbm, idx_v, ring_v, sg, sw):
        wid = lax.axis_index("s") * nc + lax.axis_index("c")
        b0 = wid * bpw
        pltpu.sync_copy(x_hbm.at[pl.ds(b0 * seq, rpw)], idx_v)

        def gather(u):
            return pltpu.async_copy(
                table_hbm.at[idx_v.at[pl.ds(u * UNIT, UNIT)]],
                ring_v.at[pl.ds((u % RING) * UNIT, UNIT)],
                sg[u % RING],
            )

        pend_g = [gather(u) for u in range(RING)]
        issued = RING
        waited_g = 0
        waited_w = -1
        pend_w = [None] * bpw
        for k in range(bpw):
            last_u = (seq * k + seq - 1) // UNIT
            while waited_g <= last_u:
                pend_g[waited_g].wait()
                waited_g += 1
            pend_w[k] = pltpu.async_copy(
                ring_v.at[pl.ds((seq * k) % ring_rows, seq)],
                out_hbm.at[b0 + k],
                sw[k % 2],
            )
            while issued < nunit:
                wt = (UNIT * (issued - RING) + UNIT - 1) // seq
                if wt > k:
                    break
                while waited_w < wt:
                    waited_w += 1
                    pend_w[waited_w].wait()
                pend_g.append(gather(issued))
                issued += 1
        for k2 in range(waited_w + 1, bpw):
            pend_w[k2].wait()

    return emb


def kernel(x, table):
    emb = _build(BATCH, SEQ, EMBED)
    return emb(x.reshape(-1), table)
